# Initial kernel scaffold; baseline (speedup 1.0000x reference)
import jax
import jax.numpy as jnp
from jax.experimental import pallas as pl


def kernel(x, edge_index, edge_attr, start_right, W_msg_0, W_edge_0, W_self_0, W_msg_1, W_edge_1, W_self_1):
    raise NotImplementedError("write your pallas kernel here")


# speedup vs baseline: 2.6653x; 2.6653x over previous
import functools

import jax
import jax.numpy as jnp
from jax import lax
from jax.experimental import pallas as pl
from jax.experimental.pallas import tpu as pltpu
from jax.experimental.pallas import tpu_sc as plsc

N_NODES = 10000
N_HALF = 5000
D_FEAT = 128
D_EDGE = 16
N_EDGES = 320000

NC = 2
NS = 16
NW = NC * NS
CHUNK = 128
CHUNKS_PER_W = -(-N_EDGES // (NW * CHUNK))
NE_PAD = NW * CHUNK * CHUNKS_PER_W
ACC_ROWS = 5008
ROWS_PER_TILE = 312
TAIL_ROW0 = NS * ROWS_PER_TILE
TAIL = ACC_ROWS - TAIL_ROW0
PIECES = (64, 64, 64, 64, 56)


@functools.partial(
    pl.kernel,
    out_type=[
        jax.ShapeDtypeStruct((NC * ACC_ROWS, D_FEAT), jnp.float32),
        jax.ShapeDtypeStruct((NC * ACC_ROWS, D_FEAT), jnp.float32),
    ],
    mesh=plsc.VectorSubcoreMesh(core_axis_name="c", subcore_axis_name="s"),
    scratch_types=[
        pltpu.VMEM((CHUNK,), jnp.int32),
        pltpu.VMEM((CHUNK,), jnp.int32),
        pltpu.VMEM((CHUNK, D_FEAT), jnp.float32),
        pltpu.VMEM((CHUNK, D_EDGE), jnp.float32),
        pltpu.VMEM((CHUNK, D_FEAT), jnp.float32),
        pltpu.VMEM_SHARED((ACC_ROWS, D_FEAT), jnp.float32),
        pltpu.VMEM_SHARED((ACC_ROWS, D_FEAT), jnp.float32),
        pltpu.SemaphoreType.DMA,
    ],
)
def _sc_pass(x_hbm, gidx_hbm, sidx_hbm, ea_hbm, zg_hbm,
             outg_hbm, oute_hbm,
             gidx_v, sidx_v, rows_v, e16_v, e128_v, g_acc, e_acc, sem):
    c = lax.axis_index("c")
    s = lax.axis_index("s")
    wid = s * NC + c
    row0 = s * ROWS_PER_TILE

    pltpu.sync_copy(zg_hbm, e128_v)
    off = 0
    for p in PIECES:
        pltpu.sync_copy(e128_v.at[pl.ds(0, p)],
                        g_acc.at[pl.ds(row0 + off, p)])
        pltpu.sync_copy(e128_v.at[pl.ds(0, p)],
                        e_acc.at[pl.ds(row0 + off, p)])
        off += p

    @pl.when(s == NS - 1)
    def _zero_tail():
        pltpu.sync_copy(e128_v.at[pl.ds(0, TAIL)],
                        g_acc.at[pl.ds(TAIL_ROW0, TAIL)])
        pltpu.sync_copy(e128_v.at[pl.ds(0, TAIL)],
                        e_acc.at[pl.ds(TAIL_ROW0, TAIL)])

    plsc.subcore_barrier()

    @pl.loop(0, CHUNKS_PER_W)
    def body(t):
        base = (wid * CHUNKS_PER_W + t) * CHUNK
        pltpu.sync_copy(gidx_hbm.at[pl.ds(base, CHUNK)], gidx_v)
        pltpu.sync_copy(sidx_hbm.at[pl.ds(base, CHUNK)], sidx_v)
        pltpu.sync_copy(ea_hbm.at[pl.ds(base, CHUNK)], e16_v)
        pltpu.async_copy(x_hbm.at[gidx_v], rows_v, sem).wait()

        @pl.loop(0, CHUNK, unroll=16)
        def expand(e):
            e128_v[e, pl.ds(0, D_EDGE)] = e16_v[e, :]

        pltpu.sync_copy(rows_v, g_acc.at[sidx_v], add=True)
        pltpu.sync_copy(e128_v, e_acc.at[sidx_v], add=True)

    plsc.subcore_barrier()

    out_row0 = c * ACC_ROWS + row0
    off = 0
    for p in PIECES:
        pltpu.sync_copy(g_acc.at[pl.ds(row0 + off, p)], rows_v.at[pl.ds(0, p)])
        pltpu.sync_copy(rows_v.at[pl.ds(0, p)],
                        outg_hbm.at[pl.ds(out_row0 + off, p)])
        pltpu.sync_copy(e_acc.at[pl.ds(row0 + off, p)], rows_v.at[pl.ds(0, p)])
        pltpu.sync_copy(rows_v.at[pl.ds(0, p)],
                        oute_hbm.at[pl.ds(out_row0 + off, p)])
        off += p

    @pl.when(s == NS - 1)
    def _out_tail():
        pltpu.sync_copy(g_acc.at[pl.ds(TAIL_ROW0, TAIL)],
                        rows_v.at[pl.ds(0, TAIL)])
        pltpu.sync_copy(rows_v.at[pl.ds(0, TAIL)],
                        outg_hbm.at[pl.ds(c * ACC_ROWS + TAIL_ROW0, TAIL)])
        pltpu.sync_copy(e_acc.at[pl.ds(TAIL_ROW0, TAIL)],
                        rows_v.at[pl.ds(0, TAIL)])
        pltpu.sync_copy(rows_v.at[pl.ds(0, TAIL)],
                        oute_hbm.at[pl.ds(c * ACC_ROWS + TAIL_ROW0, TAIL)])


BLK = 1000


def _tc_active_body(x_ref, gp_ref, ep_ref, ws_ref, wm_ref, we_ref, o_ref):
    g = gp_ref[0] + gp_ref[1]
    e = ep_ref[0] + ep_ref[1]
    acc = jnp.dot(x_ref[...], ws_ref[...], preferred_element_type=jnp.float32)
    acc = acc + jnp.dot(g, wm_ref[...], preferred_element_type=jnp.float32)
    acc = acc + jnp.dot(e, we_ref[...], preferred_element_type=jnp.float32)
    o_ref[...] = jnp.maximum(acc, 0.0)


def _tc_active(x, gp, ep, ws, wm, we):
    return pl.pallas_call(
        _tc_active_body,
        grid=(N_HALF // BLK,),
        in_specs=[
            pl.BlockSpec((BLK, D_FEAT), lambda i: (i, 0)),
            pl.BlockSpec((NC, BLK, D_FEAT), lambda i: (0, i, 0)),
            pl.BlockSpec((NC, BLK, D_EDGE), lambda i: (0, i, 0)),
            pl.BlockSpec((D_FEAT, D_FEAT), lambda i: (0, 0)),
            pl.BlockSpec((D_FEAT, D_FEAT), lambda i: (0, 0)),
            pl.BlockSpec((D_EDGE, D_FEAT), lambda i: (0, 0)),
        ],
        out_specs=pl.BlockSpec((BLK, D_FEAT), lambda i: (i, 0)),
        out_shape=jax.ShapeDtypeStruct((N_HALF, D_FEAT), jnp.float32),
    )(x, gp, ep, ws, wm, we)


def _tc_passive_body(x_ref, ws_ref, o_ref):
    acc = jnp.dot(x_ref[...], ws_ref[...], preferred_element_type=jnp.float32)
    o_ref[...] = jnp.maximum(acc, 0.0)


def _tc_passive(x, ws):
    return pl.pallas_call(
        _tc_passive_body,
        grid=(N_HALF // BLK,),
        in_specs=[
            pl.BlockSpec((BLK, D_FEAT), lambda i: (i, 0)),
            pl.BlockSpec((D_FEAT, D_FEAT), lambda i: (0, 0)),
        ],
        out_specs=pl.BlockSpec((BLK, D_FEAT), lambda i: (i, 0)),
        out_shape=jax.ShapeDtypeStruct((N_HALF, D_FEAT), jnp.float32),
    )(x, ws)


def _layer(x, gidx, sidx, ea, zg, active_right, W_msg, W_edge, W_self):
    g, e = _sc_pass(x, gidx, sidx, ea, zg)
    gp = g.reshape(NC, ACC_ROWS, D_FEAT)[:, :N_HALF]
    ep = e.reshape(NC, ACC_ROWS, D_FEAT)[:, :N_HALF, :D_EDGE]
    if active_right:
        act = _tc_active(x[N_HALF:], gp, ep, W_self, W_msg, W_edge)
        pas = _tc_passive(x[:N_HALF], W_self)
        return jnp.concatenate([pas, act], axis=0)
    act = _tc_active(x[:N_HALF], gp, ep, W_self, W_msg, W_edge)
    pas = _tc_passive(x[N_HALF:], W_self)
    return jnp.concatenate([act, pas], axis=0)


def kernel(x, edge_index, edge_attr, start_right,
           W_msg_0, W_edge_0, W_self_0,
           W_msg_1, W_edge_1, W_self_1):
    src = edge_index[0]
    dst = edge_index[1]
    pad = NE_PAD - N_EDGES
    pad_g = jnp.zeros((pad,), jnp.int32)
    pad_s = jnp.full((pad,), N_HALF, jnp.int32)
    gidx0 = jnp.concatenate([src, pad_g])
    sidx0 = jnp.concatenate([dst - N_HALF, pad_s])
    gidx1 = jnp.concatenate([dst, pad_g])
    sidx1 = jnp.concatenate([src, pad_s])
    ea = jnp.concatenate([edge_attr, jnp.zeros((pad, D_EDGE), jnp.float32)])
    zg = jnp.zeros((CHUNK, D_FEAT), jnp.float32)

    x1 = _layer(x, gidx0, sidx0, ea, zg, True, W_msg_0, W_edge_0, W_self_0)
    x2 = _layer(x1, gidx1, sidx1, ea, zg, False, W_msg_1, W_edge_1, W_self_1)
    return x2
